# baseline (speedup 1.0000x reference)
import functools

import jax
import jax.numpy as jnp
from jax import lax
from jax.experimental import pallas as pl
from jax.experimental.pallas import tpu as pltpu
from jax.experimental.pallas import tpu_sc as plsc

N = 10000
E = 320000
D = 128
H = 12
HP = 16
NEG = -1e30

NC = 2
NS = 16
NW = NC * NS
E_W = E // NW
KB = 400
NBLK = E_W // KB
KS = 80
NBLKS = E_W // KS
ZR = 40
KE = 1000
GE = E // KE


def _mesh():
    return plsc.VectorSubcoreMesh(core_axis_name="c", subcore_axis_name="s")


def _gatherx_body(xb_hbm, src_hbm, xj_hbm, srcv, xjv, sg, so):
    wid = lax.axis_index("s") * NC + lax.axis_index("c")
    base = wid * E_W

    def fetch(b, s):
        off = base + b * KB
        pltpu.sync_copy(src_hbm.at[pl.ds(off, KB)], srcv.at[pl.ds(s * KB, KB)])
        pltpu.async_copy(xb_hbm.at[srcv.at[pl.ds(s * KB, KB)]], xjv.at[s], sg.at[s])

    def wait_fetch(s):
        pltpu.make_async_copy(xb_hbm.at[srcv.at[pl.ds(s * KB, KB)]], xjv.at[s],
                              sg.at[s]).wait()

    def wait_out(s):
        pltpu.make_async_copy(xjv.at[s], xj_hbm.at[pl.ds(0, KB)], so.at[s]).wait()

    fetch(0, 0)

    def blk(b, _):
        s = lax.rem(b, 2)
        s2 = 1 - s

        @pl.when(b + 1 < NBLK)
        def _():
            @pl.when(b >= 1)
            def _():
                wait_out(s2)
            fetch(b + 1, s2)

        wait_fetch(s)
        off = base + b * KB
        pltpu.async_copy(xjv.at[s], xj_hbm.at[pl.ds(off, KB)], so.at[s])
        return 0

    lax.fori_loop(0, NBLK, blk, 0)
    wait_out(0)
    wait_out(1)


def _sc_gather_x(xb, src):
    k = pl.kernel(
        _gatherx_body,
        out_type=jax.ShapeDtypeStruct((E, D), jnp.float32),
        mesh=_mesh(),
        scratch_types=[
            pltpu.VMEM((2 * KB,), jnp.int32),
            pltpu.VMEM((2, KB, D), jnp.float32),
            pltpu.SemaphoreType.DMA((2,)),
            pltpu.SemaphoreType.DMA((2,)),
        ],
    )
    return k(xb, src)


def _gathera_body(a_hbm, src_hbm, dst_hbm, l_hbm,
                  srcv, dstv, ajv, aiv, lv, sg, so):
    wid = lax.axis_index("s") * NC + lax.axis_index("c")
    base = wid * E_W
    lane = lax.iota(jnp.int32, HP)
    is12 = lane == 12

    def fetch(b, s):
        off = base + b * KB
        pltpu.sync_copy(src_hbm.at[pl.ds(off, KB)], srcv.at[s, pl.ds(0, KB)])
        pltpu.sync_copy(dst_hbm.at[pl.ds(off, KB)], dstv.at[s, pl.ds(0, KB)])
        pltpu.async_copy(a_hbm.at[srcv.at[s, pl.ds(0, KB)]], ajv.at[s], sg.at[s])
        pltpu.async_copy(a_hbm.at[dstv.at[s, pl.ds(0, KB)]], aiv.at[s], sg.at[s])

    def wait_fetch(s):
        pltpu.make_async_copy(a_hbm.at[srcv.at[s, pl.ds(0, KB)]], ajv.at[s], sg.at[s]).wait()
        pltpu.make_async_copy(a_hbm.at[dstv.at[s, pl.ds(0, KB)]], aiv.at[s], sg.at[s]).wait()

    def wait_out(s):
        pltpu.make_async_copy(lv.at[s], l_hbm.at[pl.ds(0, KB)], so.at[s]).wait()

    fetch(0, 0)

    def blk(b, _):
        s = lax.rem(b, 2)
        s2 = 1 - s

        @pl.when(b + 1 < NBLK)
        def _():
            @pl.when(b >= 1)
            def _():
                wait_out(s2)
            fetch(b + 1, s2)

        wait_fetch(s)
        srcv_s, dstv_s = srcv.at[s], dstv.at[s]
        aiv_s, ajv_s, lv_s = aiv.at[s], ajv.at[s], lv.at[s]

        def grp16(g, nj):
            sv = srcv_s[pl.ds(g * 16, 16)]
            dv = dstv_s[pl.ds(g * 16, 16)]
            wv = jnp.where(sv != dv, 1.0, 0.0).astype(jnp.float32)
            for j in range(nj):
                e = g * 16 + j
                l = aiv_s[e, :] - ajv_s[e, :]
                lv_s[e, :] = jnp.where(is12, wv[j], l)

        def grp(g, _):
            grp16(g, 16)
            return 0

        lax.fori_loop(0, KB // 16, grp, 0)
        if KB % 16:
            grp16(KB // 16, KB % 16)
        off = base + b * KB
        pltpu.async_copy(lv.at[s], l_hbm.at[pl.ds(off, KB)], so.at[s])
        return 0

    lax.fori_loop(0, NBLK, blk, 0)
    wait_out(0)
    wait_out(1)


def _sc_gather_a(a, src, dst):
    k = pl.kernel(
        _gathera_body,
        out_type=jax.ShapeDtypeStruct((E, HP), jnp.float32),
        mesh=_mesh(),
        compiler_params=pltpu.CompilerParams(use_tc_tiling_on_sc=False),
        scratch_types=[
            pltpu.VMEM((2, KB + 16), jnp.int32),
            pltpu.VMEM((2, KB + 16), jnp.int32),
            pltpu.VMEM((2, KB, HP), jnp.float32),
            pltpu.VMEM((2, KB, HP), jnp.float32),
            pltpu.VMEM((2, KB, HP), jnp.float32),
            pltpu.SemaphoreType.DMA((2,)),
            pltpu.SemaphoreType.DMA((2,)),
        ],
    )
    return k(a, src, dst)


def _scatter_body(with_cnt, msg_hbm, wrow_hbm, dst_hbm, acc_out, cnt_out,
                  msgv, dstv, wrowv, zbuf, zbufc, acc_sh, cnt_sh, sl):
    cid = lax.axis_index("c")
    sid = lax.axis_index("s")
    wid = sid * NC + cid
    base = wid * E_W

    def zrow(i, _):
        r = i // (D // HP)
        c = i % (D // HP)
        zbuf[r, pl.ds(c * HP, HP)] = jnp.zeros((HP,), jnp.float32)
        return 0

    lax.fori_loop(0, ZR * (D // HP), zrow, 0)

    if with_cnt:
        def zrowc(i, _):
            zbufc[i, :] = jnp.zeros((HP,), jnp.float32)
            return 0
        lax.fori_loop(0, ZR, zrowc, 0)

    nzb = N // ZR

    def zcp(k, _):
        b = sid + k * NS

        @pl.when(b < nzb)
        def _():
            pltpu.sync_copy(zbuf, acc_sh.at[pl.ds(b * ZR, ZR)])
            if with_cnt:
                pltpu.sync_copy(zbufc, cnt_sh.at[pl.ds(b * ZR, ZR)])
        return 0

    lax.fori_loop(0, (nzb + NS - 1) // NS, zcp, 0)
    plsc.subcore_barrier()

    def fetch(b, s):
        off = base + b * KS
        pltpu.async_copy(msg_hbm.at[pl.ds(off, KS)], msgv.at[s], sl.at[s])
        pltpu.async_copy(dst_hbm.at[pl.ds(off, KS)], dstv.at[s], sl.at[s])
        if with_cnt:
            pltpu.async_copy(wrow_hbm.at[pl.ds(off, KS)], wrowv.at[s], sl.at[s])

    def wait_fetch(s):
        pltpu.make_async_copy(msg_hbm.at[pl.ds(0, KS)], msgv.at[s], sl.at[s]).wait()
        pltpu.make_async_copy(dst_hbm.at[pl.ds(0, KS)], dstv.at[s], sl.at[s]).wait()
        if with_cnt:
            pltpu.make_async_copy(wrow_hbm.at[pl.ds(0, KS)], wrowv.at[s], sl.at[s]).wait()

    fetch(0, 0)

    def blk(b, _):
        s = lax.rem(b, 2)

        @pl.when(b + 1 < NBLKS)
        def _():
            fetch(b + 1, 1 - s)

        wait_fetch(s)
        pltpu.sync_copy(msgv.at[s], acc_sh.at[dstv.at[s]], add=True)
        if with_cnt:
            pltpu.sync_copy(wrowv.at[s], cnt_sh.at[dstv.at[s]], add=True)
        return 0

    lax.fori_loop(0, NBLKS, blk, 0)
    plsc.subcore_barrier()

    @pl.when(sid == 0)
    def _():
        pltpu.sync_copy(acc_sh, acc_out.at[cid])
        if with_cnt:
            pltpu.sync_copy(cnt_sh, cnt_out.at[cid])


def _sc_scatter(msg, wrow, dst, with_cnt):
    k = pl.kernel(
        functools.partial(_scatter_body, with_cnt),
        out_type=(jax.ShapeDtypeStruct((NC, N, D), jnp.float32),
                  jax.ShapeDtypeStruct((NC, N, HP), jnp.float32)),
        mesh=_mesh(),
        compiler_params=pltpu.CompilerParams(use_tc_tiling_on_sc=False),
        scratch_types=[
            pltpu.VMEM((2, KS, D), jnp.float32),
            pltpu.VMEM((2, KS), jnp.int32),
            pltpu.VMEM((2, KS, HP), jnp.float32),
            pltpu.VMEM((ZR, D), jnp.float32),
            pltpu.VMEM((ZR, HP), jnp.float32),
            pltpu.VMEM_SHARED((N, D), jnp.float32),
            pltpu.VMEM_SHARED((N, HP), jnp.float32),
            pltpu.SemaphoreType.DMA((2,)),
        ],
    )
    return k(msg, wrow, dst)


def _wc_from(wl, cp):
    m = jnp.max(cp, axis=1, keepdims=True)
    p = jnp.exp(cp - m)
    sc = p / jnp.sum(p, axis=1, keepdims=True)
    wc = jnp.zeros((D, D), jnp.float32)
    for h in range(H):
        wc = wc + sc[0, h] * wl[h * D:(h + 1) * D, :]
    return wc


def _prep_kern(x_ref, wu_ref, wl_ref, cp_ref, a_ref, b_ref):
    x = x_ref[...]
    a_ref[...] = lax.dot_general(x, wu_ref[...], (((1,), (1,)), ((), ())),
                                 preferred_element_type=jnp.float32)
    wc = _wc_from(wl_ref[...], cp_ref[...])
    b_ref[...] = lax.dot_general(x, wc, (((1,), (1,)), ((), ())),
                                 preferred_element_type=jnp.float32)


def _tc_prep(x, wu16, wlin, cp):
    nb = 10
    rows = N // nb
    return pl.pallas_call(
        _prep_kern,
        grid=(nb,),
        in_specs=[
            pl.BlockSpec((rows, D), lambda i: (i, 0)),
            pl.BlockSpec((HP, D), lambda i: (0, 0)),
            pl.BlockSpec((H * D, D), lambda i: (0, 0)),
            pl.BlockSpec((1, HP), lambda i: (0, 0)),
        ],
        out_specs=[
            pl.BlockSpec((rows, HP), lambda i: (i, 0)),
            pl.BlockSpec((rows, D), lambda i: (i, 0)),
        ],
        out_shape=[
            jax.ShapeDtypeStruct((N, HP), jnp.float32),
            jax.ShapeDtypeStruct((N, D), jnp.float32),
        ],
    )(x, wu16, wlin, cp)


def _edge_kern(with_w, xj_ref, l_ref, wl_ref, cp_ref, msg_ref, *rest):
    l = l_ref[...]
    logits = l + cp_ref[...]
    m = jnp.max(logits, axis=1, keepdims=True)
    p = jnp.exp(logits - m)
    att = p / jnp.sum(p, axis=1, keepdims=True)
    w = l[:, 12:13]
    attw = att * w
    xj = xj_ref[...].astype(jnp.bfloat16)
    xt = lax.dot_general(xj, wl_ref[...], (((1,), (1,)), ((), ())),
                         preferred_element_type=jnp.float32)
    acc = attw[:, 0:1] * xt[:, 0:D]
    for h in range(1, H):
        acc = acc + attw[:, h:h + 1] * xt[:, h * D:(h + 1) * D]
    msg_ref[...] = acc
    if with_w:
        rest[0][...] = jnp.broadcast_to(w, (KE, HP))


def _tc_edge(xj, l, wlin_bf, cp, with_w):
    out_specs = [pl.BlockSpec((KE, D), lambda i: (i, 0))]
    out_shape = [jax.ShapeDtypeStruct((E, D), jnp.float32)]
    if with_w:
        out_specs.append(pl.BlockSpec((KE, HP), lambda i: (i, 0)))
        out_shape.append(jax.ShapeDtypeStruct((E, HP), jnp.float32))
    return pl.pallas_call(
        functools.partial(_edge_kern, with_w),
        grid=(GE,),
        in_specs=[
            pl.BlockSpec((KE, D), lambda i: (i, 0)),
            pl.BlockSpec((KE, HP), lambda i: (i, 0)),
            pl.BlockSpec((H * D, D), lambda i: (0, 0)),
            pl.BlockSpec((1, HP), lambda i: (0, 0)),
        ],
        out_specs=out_specs,
        out_shape=out_shape,
    )(xj, l, wlin_bf, cp)


def _post1_kern(acc_ref, cnt_ref, bself_ref, b1_ref, hpre_ref, st_ref):
    i = pl.program_id(0)
    summed = acc_ref[0] + acc_ref[1] + bself_ref[...]
    cnt = cnt_ref[0, :, 0:1] + cnt_ref[1, :, 0:1] + 1.0
    hpre = summed / jnp.maximum(cnt, 1.0) + b1_ref[...]
    hpre = jnp.maximum(hpre, 0.0)
    hpre_ref[...] = hpre

    @pl.when(i == 0)
    def _():
        st_ref[...] = jnp.zeros_like(st_ref)

    s1 = jnp.sum(hpre, axis=0, keepdims=True)
    s2 = jnp.sum(hpre * hpre, axis=0, keepdims=True)
    st_ref[0:1, :] += s1
    st_ref[1:2, :] += s2


def _tc_post1(acc, cnt, bself, b1row):
    nb = 10
    rows = N // nb
    return pl.pallas_call(
        _post1_kern,
        grid=(nb,),
        in_specs=[
            pl.BlockSpec((NC, rows, D), lambda i: (0, i, 0)),
            pl.BlockSpec((NC, rows, HP), lambda i: (0, i, 0)),
            pl.BlockSpec((rows, D), lambda i: (i, 0)),
            pl.BlockSpec((1, D), lambda i: (0, 0)),
        ],
        out_specs=[
            pl.BlockSpec((rows, D), lambda i: (i, 0)),
            pl.BlockSpec((8, D), lambda i: (0, 0)),
        ],
        out_shape=[
            jax.ShapeDtypeStruct((N, D), jnp.float32),
            jax.ShapeDtypeStruct((8, D), jnp.float32),
        ],
    )(acc, cnt, bself, b1row)


def _bn_kern(hpre_ref, st_ref, g_ref, bt_ref, wu_ref, wl_ref, cp_ref,
             a_ref, b_ref, hb_ref):
    mean = st_ref[0:1, :] / float(N)
    var = st_ref[1:2, :] / float(N) - mean * mean
    h = (hpre_ref[...] - mean) * lax.rsqrt(var + 1e-5) * g_ref[...] + bt_ref[...]
    a_ref[...] = lax.dot_general(h, wu_ref[...], (((1,), (1,)), ((), ())),
                                 preferred_element_type=jnp.float32)
    wc = _wc_from(wl_ref[...], cp_ref[...])
    b_ref[...] = lax.dot_general(h, wc, (((1,), (1,)), ((), ())),
                                 preferred_element_type=jnp.float32)
    hb_ref[...] = h


def _tc_bn_prep(hpre, st, gamma, beta, wu16, wlin, cp):
    nb = 10
    rows = N // nb
    return pl.pallas_call(
        _bn_kern,
        grid=(nb,),
        in_specs=[
            pl.BlockSpec((rows, D), lambda i: (i, 0)),
            pl.BlockSpec((8, D), lambda i: (0, 0)),
            pl.BlockSpec((1, D), lambda i: (0, 0)),
            pl.BlockSpec((1, D), lambda i: (0, 0)),
            pl.BlockSpec((HP, D), lambda i: (0, 0)),
            pl.BlockSpec((H * D, D), lambda i: (0, 0)),
            pl.BlockSpec((1, HP), lambda i: (0, 0)),
        ],
        out_specs=[
            pl.BlockSpec((rows, HP), lambda i: (i, 0)),
            pl.BlockSpec((rows, D), lambda i: (i, 0)),
            pl.BlockSpec((rows, D), lambda i: (i, 0)),
        ],
        out_shape=[
            jax.ShapeDtypeStruct((N, HP), jnp.float32),
            jax.ShapeDtypeStruct((N, D), jnp.float32),
            jax.ShapeDtypeStruct((N, D), jnp.float32),
        ],
    )(hpre, st, gamma, beta, wu16, wlin, cp)


def _post2_kern(acc_ref, cnt_ref, bself_ref, b2_ref, out_ref):
    summed = acc_ref[0] + acc_ref[1] + bself_ref[...]
    cnt = cnt_ref[0, :, 0:1] + cnt_ref[1, :, 0:1] + 1.0
    out_ref[...] = summed / jnp.maximum(cnt, 1.0) + b2_ref[...]


def _tc_post2(acc, cnt, bself, b2row):
    nb = 10
    rows = N // nb
    return pl.pallas_call(
        _post2_kern,
        grid=(nb,),
        in_specs=[
            pl.BlockSpec((NC, rows, D), lambda i: (0, i, 0)),
            pl.BlockSpec((NC, rows, HP), lambda i: (0, i, 0)),
            pl.BlockSpec((rows, D), lambda i: (i, 0)),
            pl.BlockSpec((1, D), lambda i: (0, 0)),
        ],
        out_specs=pl.BlockSpec((rows, D), lambda i: (i, 0)),
        out_shape=jax.ShapeDtypeStruct((N, D), jnp.float32),
    )(acc, cnt, bself, b2row)


def kernel(x, edge_index, W1_lin, W1_u, c1, b1, bn_gamma, bn_beta,
           W2_lin, W2_u, c2, b2):
    src = edge_index[0].astype(jnp.int32)
    dst = edge_index[1].astype(jnp.int32)

    w1u16 = jnp.pad(W1_u, ((0, HP - H), (0, 0)))
    w2u16 = jnp.pad(W2_u, ((0, HP - H), (0, 0)))
    c1p = jnp.pad(c1, (0, HP - H), constant_values=NEG).reshape(1, HP)
    c2p = jnp.pad(c2, (0, HP - H), constant_values=NEG).reshape(1, HP)
    w1l_bf = W1_lin.astype(jnp.bfloat16)
    w2l_bf = W2_lin.astype(jnp.bfloat16)
    b1r = b1.reshape(1, D)
    b2r = b2.reshape(1, D)

    a1, bs1 = _tc_prep(x, w1u16, W1_lin, c1p)
    xj1 = _sc_gather_x(x, src)
    l1 = _sc_gather_a(a1, src, dst)
    msg1, wrow = _tc_edge(xj1, l1, w1l_bf, c1p, True)
    acc1, cnt = _sc_scatter(msg1, wrow, dst, True)
    hpre, st = _tc_post1(acc1, cnt, bs1, b1r)
    a2, bs2, h = _tc_bn_prep(hpre, st, bn_gamma.reshape(1, D),
                             bn_beta.reshape(1, D), w2u16, W2_lin, c2p)

    xj2 = _sc_gather_x(h, src)
    l2 = _sc_gather_a(a2, src, dst)
    (msg2,) = _tc_edge(xj2, l2, w2l_bf, c2p, False)
    acc2, _ = _sc_scatter(msg2, wrow, dst, False)
    return _tc_post2(acc2, cnt, bs2, b2r)


# speedup vs baseline: 1.0013x; 1.0013x over previous
import functools

import jax
import jax.numpy as jnp
from jax import lax
from jax.experimental import pallas as pl
from jax.experimental.pallas import tpu as pltpu
from jax.experimental.pallas import tpu_sc as plsc

N = 10000
E = 320000
D = 128
H = 12
HP = 16
NEG = -1e30

NC = 2
NS = 16
NW = NC * NS
E_W = E // NW
KB = 400
NBLK = E_W // KB
KS = 80
NBLKS = E_W // KS
ZR = 40
KE = 1000
GE = E // KE


def _mesh():
    return plsc.VectorSubcoreMesh(core_axis_name="c", subcore_axis_name="s")


def _gatherx_body(xb_hbm, src_hbm, xj_hbm, srcv, xjv, sg, so):
    wid = lax.axis_index("s") * NC + lax.axis_index("c")
    base = wid * E_W

    def fetch(b, s):
        off = base + b * KB
        pltpu.sync_copy(src_hbm.at[pl.ds(off, KB)], srcv.at[pl.ds(s * KB, KB)])
        pltpu.async_copy(xb_hbm.at[srcv.at[pl.ds(s * KB, KB)]], xjv.at[s], sg.at[s])

    def wait_fetch(s):
        pltpu.make_async_copy(xb_hbm.at[srcv.at[pl.ds(s * KB, KB)]], xjv.at[s],
                              sg.at[s]).wait()

    def wait_out(s):
        pltpu.make_async_copy(xjv.at[s], xj_hbm.at[pl.ds(0, KB)], so.at[s]).wait()

    fetch(0, 0)

    def blk(b, _):
        s = lax.rem(b, 2)
        s2 = 1 - s

        @pl.when(b + 1 < NBLK)
        def _():
            @pl.when(b >= 1)
            def _():
                wait_out(s2)
            fetch(b + 1, s2)

        wait_fetch(s)
        off = base + b * KB
        pltpu.async_copy(xjv.at[s], xj_hbm.at[pl.ds(off, KB)], so.at[s])
        return 0

    lax.fori_loop(0, NBLK, blk, 0)
    wait_out(0)
    wait_out(1)


def _sc_gather_x(xb, src):
    k = pl.kernel(
        _gatherx_body,
        out_type=jax.ShapeDtypeStruct((E, D), jnp.float32),
        mesh=_mesh(),
        compiler_params=pltpu.CompilerParams(use_tc_tiling_on_sc=False),
        scratch_types=[
            pltpu.VMEM((2 * KB,), jnp.int32),
            pltpu.VMEM((2, KB, D), jnp.float32),
            pltpu.SemaphoreType.DMA((2,)),
            pltpu.SemaphoreType.DMA((2,)),
        ],
    )
    return k(xb, src)


def _gathera_body(a_hbm, src_hbm, dst_hbm, l_hbm,
                  srcv, dstv, ajv, aiv, lv, sg, so):
    wid = lax.axis_index("s") * NC + lax.axis_index("c")
    base = wid * E_W
    lane = lax.iota(jnp.int32, HP)
    is12 = lane == 12

    def fetch(b, s):
        off = base + b * KB
        pltpu.sync_copy(src_hbm.at[pl.ds(off, KB)], srcv.at[s, pl.ds(0, KB)])
        pltpu.sync_copy(dst_hbm.at[pl.ds(off, KB)], dstv.at[s, pl.ds(0, KB)])
        pltpu.async_copy(a_hbm.at[srcv.at[s, pl.ds(0, KB)]], ajv.at[s], sg.at[s])
        pltpu.async_copy(a_hbm.at[dstv.at[s, pl.ds(0, KB)]], aiv.at[s], sg.at[s])

    def wait_fetch(s):
        pltpu.make_async_copy(a_hbm.at[srcv.at[s, pl.ds(0, KB)]], ajv.at[s], sg.at[s]).wait()
        pltpu.make_async_copy(a_hbm.at[dstv.at[s, pl.ds(0, KB)]], aiv.at[s], sg.at[s]).wait()

    def wait_out(s):
        pltpu.make_async_copy(lv.at[s], l_hbm.at[pl.ds(0, KB)], so.at[s]).wait()

    fetch(0, 0)

    def blk(b, _):
        s = lax.rem(b, 2)
        s2 = 1 - s

        @pl.when(b + 1 < NBLK)
        def _():
            @pl.when(b >= 1)
            def _():
                wait_out(s2)
            fetch(b + 1, s2)

        wait_fetch(s)
        srcv_s, dstv_s = srcv.at[s], dstv.at[s]
        aiv_s, ajv_s, lv_s = aiv.at[s], ajv.at[s], lv.at[s]

        def grp16(g, nj):
            sv = srcv_s[pl.ds(g * 16, 16)]
            dv = dstv_s[pl.ds(g * 16, 16)]
            wv = jnp.where(sv != dv, 1.0, 0.0).astype(jnp.float32)
            for j in range(nj):
                e = g * 16 + j
                l = aiv_s[e, :] - ajv_s[e, :]
                lv_s[e, :] = jnp.where(is12, wv[j], l)

        def grp(g, _):
            grp16(g, 16)
            return 0

        lax.fori_loop(0, KB // 16, grp, 0)
        if KB % 16:
            grp16(KB // 16, KB % 16)
        off = base + b * KB
        pltpu.async_copy(lv.at[s], l_hbm.at[pl.ds(off, KB)], so.at[s])
        return 0

    lax.fori_loop(0, NBLK, blk, 0)
    wait_out(0)
    wait_out(1)


def _sc_gather_a(a, src, dst):
    k = pl.kernel(
        _gathera_body,
        out_type=jax.ShapeDtypeStruct((E, HP), jnp.float32),
        mesh=_mesh(),
        compiler_params=pltpu.CompilerParams(use_tc_tiling_on_sc=False),
        scratch_types=[
            pltpu.VMEM((2, KB + 16), jnp.int32),
            pltpu.VMEM((2, KB + 16), jnp.int32),
            pltpu.VMEM((2, KB, HP), jnp.float32),
            pltpu.VMEM((2, KB, HP), jnp.float32),
            pltpu.VMEM((2, KB, HP), jnp.float32),
            pltpu.SemaphoreType.DMA((2,)),
            pltpu.SemaphoreType.DMA((2,)),
        ],
    )
    return k(a, src, dst)


def _scatter_body(with_cnt, msg_hbm, wrow_hbm, dst_hbm, acc_out, cnt_out,
                  msgv, dstv, wrowv, zbuf, zbufc, acc_sh, cnt_sh, sl):
    cid = lax.axis_index("c")
    sid = lax.axis_index("s")
    wid = sid * NC + cid
    base = wid * E_W

    def zrow(i, _):
        r = i // (D // HP)
        c = i % (D // HP)
        zbuf[r, pl.ds(c * HP, HP)] = jnp.zeros((HP,), jnp.float32)
        return 0

    lax.fori_loop(0, ZR * (D // HP), zrow, 0)

    if with_cnt:
        def zrowc(i, _):
            zbufc[i, :] = jnp.zeros((HP,), jnp.float32)
            return 0
        lax.fori_loop(0, ZR, zrowc, 0)

    nzb = N // ZR

    def zcp(k, _):
        b = sid + k * NS

        @pl.when(b < nzb)
        def _():
            pltpu.sync_copy(zbuf, acc_sh.at[pl.ds(b * ZR, ZR)])
            if with_cnt:
                pltpu.sync_copy(zbufc, cnt_sh.at[pl.ds(b * ZR, ZR)])
        return 0

    lax.fori_loop(0, (nzb + NS - 1) // NS, zcp, 0)
    plsc.subcore_barrier()

    def fetch(b, s):
        off = base + b * KS
        pltpu.async_copy(msg_hbm.at[pl.ds(off, KS)], msgv.at[s], sl.at[s])
        pltpu.async_copy(dst_hbm.at[pl.ds(off, KS)], dstv.at[s], sl.at[s])
        if with_cnt:
            pltpu.async_copy(wrow_hbm.at[pl.ds(off, KS)], wrowv.at[s], sl.at[s])

    def wait_fetch(s):
        pltpu.make_async_copy(msg_hbm.at[pl.ds(0, KS)], msgv.at[s], sl.at[s]).wait()
        pltpu.make_async_copy(dst_hbm.at[pl.ds(0, KS)], dstv.at[s], sl.at[s]).wait()
        if with_cnt:
            pltpu.make_async_copy(wrow_hbm.at[pl.ds(0, KS)], wrowv.at[s], sl.at[s]).wait()

    fetch(0, 0)

    def blk(b, _):
        s = lax.rem(b, 2)

        @pl.when(b + 1 < NBLKS)
        def _():
            fetch(b + 1, 1 - s)

        wait_fetch(s)
        pltpu.sync_copy(msgv.at[s], acc_sh.at[dstv.at[s]], add=True)
        if with_cnt:
            pltpu.sync_copy(wrowv.at[s], cnt_sh.at[dstv.at[s]], add=True)
        return 0

    lax.fori_loop(0, NBLKS, blk, 0)
    plsc.subcore_barrier()

    @pl.when(sid == 0)
    def _():
        pltpu.sync_copy(acc_sh, acc_out.at[cid])
        if with_cnt:
            pltpu.sync_copy(cnt_sh, cnt_out.at[cid])


def _sc_scatter(msg, wrow, dst, with_cnt):
    k = pl.kernel(
        functools.partial(_scatter_body, with_cnt),
        out_type=(jax.ShapeDtypeStruct((NC, N, D), jnp.float32),
                  jax.ShapeDtypeStruct((NC, N, HP), jnp.float32)),
        mesh=_mesh(),
        compiler_params=pltpu.CompilerParams(use_tc_tiling_on_sc=False),
        scratch_types=[
            pltpu.VMEM((2, KS, D), jnp.float32),
            pltpu.VMEM((2, KS), jnp.int32),
            pltpu.VMEM((2, KS, HP), jnp.float32),
            pltpu.VMEM((ZR, D), jnp.float32),
            pltpu.VMEM((ZR, HP), jnp.float32),
            pltpu.VMEM_SHARED((N, D), jnp.float32),
            pltpu.VMEM_SHARED((N, HP), jnp.float32),
            pltpu.SemaphoreType.DMA((2,)),
        ],
    )
    return k(msg, wrow, dst)


def _wc_from(wl, cp):
    m = jnp.max(cp, axis=1, keepdims=True)
    p = jnp.exp(cp - m)
    sc = p / jnp.sum(p, axis=1, keepdims=True)
    wc = jnp.zeros((D, D), jnp.float32)
    for h in range(H):
        wc = wc + sc[0, h] * wl[h * D:(h + 1) * D, :]
    return wc


def _prep_kern(x_ref, wu_ref, wl_ref, cp_ref, a_ref, b_ref):
    x = x_ref[...]
    a_ref[...] = lax.dot_general(x, wu_ref[...], (((1,), (1,)), ((), ())),
                                 preferred_element_type=jnp.float32)
    wc = _wc_from(wl_ref[...], cp_ref[...])
    b_ref[...] = lax.dot_general(x, wc, (((1,), (1,)), ((), ())),
                                 preferred_element_type=jnp.float32)


def _tc_prep(x, wu16, wlin, cp):
    nb = 10
    rows = N // nb
    return pl.pallas_call(
        _prep_kern,
        grid=(nb,),
        in_specs=[
            pl.BlockSpec((rows, D), lambda i: (i, 0)),
            pl.BlockSpec((HP, D), lambda i: (0, 0)),
            pl.BlockSpec((H * D, D), lambda i: (0, 0)),
            pl.BlockSpec((1, HP), lambda i: (0, 0)),
        ],
        out_specs=[
            pl.BlockSpec((rows, HP), lambda i: (i, 0)),
            pl.BlockSpec((rows, D), lambda i: (i, 0)),
        ],
        out_shape=[
            jax.ShapeDtypeStruct((N, HP), jnp.float32),
            jax.ShapeDtypeStruct((N, D), jnp.float32),
        ],
    )(x, wu16, wlin, cp)


def _edge_kern(with_w, xj_ref, l_ref, wl_ref, cp_ref, msg_ref, *rest):
    l = l_ref[...]
    logits = l + cp_ref[...]
    m = jnp.max(logits, axis=1, keepdims=True)
    p = jnp.exp(logits - m)
    att = p / jnp.sum(p, axis=1, keepdims=True)
    w = l[:, 12:13]
    attw = att * w
    xj = xj_ref[...].astype(jnp.bfloat16)
    xt = lax.dot_general(xj, wl_ref[...], (((1,), (1,)), ((), ())),
                         preferred_element_type=jnp.float32)
    acc = attw[:, 0:1] * xt[:, 0:D]
    for h in range(1, H):
        acc = acc + attw[:, h:h + 1] * xt[:, h * D:(h + 1) * D]
    msg_ref[...] = acc
    if with_w:
        rest[0][...] = jnp.broadcast_to(w, (KE, HP))


def _tc_edge(xj, l, wlin_bf, cp, with_w):
    out_specs = [pl.BlockSpec((KE, D), lambda i: (i, 0))]
    out_shape = [jax.ShapeDtypeStruct((E, D), jnp.float32)]
    if with_w:
        out_specs.append(pl.BlockSpec((KE, HP), lambda i: (i, 0)))
        out_shape.append(jax.ShapeDtypeStruct((E, HP), jnp.float32))
    return pl.pallas_call(
        functools.partial(_edge_kern, with_w),
        grid=(GE,),
        in_specs=[
            pl.BlockSpec((KE, D), lambda i: (i, 0)),
            pl.BlockSpec((KE, HP), lambda i: (i, 0)),
            pl.BlockSpec((H * D, D), lambda i: (0, 0)),
            pl.BlockSpec((1, HP), lambda i: (0, 0)),
        ],
        out_specs=out_specs,
        out_shape=out_shape,
    )(xj, l, wlin_bf, cp)


def _post1_kern(acc_ref, cnt_ref, bself_ref, b1_ref, hpre_ref, st_ref):
    i = pl.program_id(0)
    summed = acc_ref[0] + acc_ref[1] + bself_ref[...]
    cnt = cnt_ref[0, :, 0:1] + cnt_ref[1, :, 0:1] + 1.0
    hpre = summed / jnp.maximum(cnt, 1.0) + b1_ref[...]
    hpre = jnp.maximum(hpre, 0.0)
    hpre_ref[...] = hpre

    @pl.when(i == 0)
    def _():
        st_ref[...] = jnp.zeros_like(st_ref)

    s1 = jnp.sum(hpre, axis=0, keepdims=True)
    s2 = jnp.sum(hpre * hpre, axis=0, keepdims=True)
    st_ref[0:1, :] += s1
    st_ref[1:2, :] += s2


def _tc_post1(acc, cnt, bself, b1row):
    nb = 10
    rows = N // nb
    return pl.pallas_call(
        _post1_kern,
        grid=(nb,),
        in_specs=[
            pl.BlockSpec((NC, rows, D), lambda i: (0, i, 0)),
            pl.BlockSpec((NC, rows, HP), lambda i: (0, i, 0)),
            pl.BlockSpec((rows, D), lambda i: (i, 0)),
            pl.BlockSpec((1, D), lambda i: (0, 0)),
        ],
        out_specs=[
            pl.BlockSpec((rows, D), lambda i: (i, 0)),
            pl.BlockSpec((8, D), lambda i: (0, 0)),
        ],
        out_shape=[
            jax.ShapeDtypeStruct((N, D), jnp.float32),
            jax.ShapeDtypeStruct((8, D), jnp.float32),
        ],
    )(acc, cnt, bself, b1row)


def _bn_kern(hpre_ref, st_ref, g_ref, bt_ref, wu_ref, wl_ref, cp_ref,
             a_ref, b_ref, hb_ref):
    mean = st_ref[0:1, :] / float(N)
    var = st_ref[1:2, :] / float(N) - mean * mean
    h = (hpre_ref[...] - mean) * lax.rsqrt(var + 1e-5) * g_ref[...] + bt_ref[...]
    a_ref[...] = lax.dot_general(h, wu_ref[...], (((1,), (1,)), ((), ())),
                                 preferred_element_type=jnp.float32)
    wc = _wc_from(wl_ref[...], cp_ref[...])
    b_ref[...] = lax.dot_general(h, wc, (((1,), (1,)), ((), ())),
                                 preferred_element_type=jnp.float32)
    hb_ref[...] = h


def _tc_bn_prep(hpre, st, gamma, beta, wu16, wlin, cp):
    nb = 10
    rows = N // nb
    return pl.pallas_call(
        _bn_kern,
        grid=(nb,),
        in_specs=[
            pl.BlockSpec((rows, D), lambda i: (i, 0)),
            pl.BlockSpec((8, D), lambda i: (0, 0)),
            pl.BlockSpec((1, D), lambda i: (0, 0)),
            pl.BlockSpec((1, D), lambda i: (0, 0)),
            pl.BlockSpec((HP, D), lambda i: (0, 0)),
            pl.BlockSpec((H * D, D), lambda i: (0, 0)),
            pl.BlockSpec((1, HP), lambda i: (0, 0)),
        ],
        out_specs=[
            pl.BlockSpec((rows, HP), lambda i: (i, 0)),
            pl.BlockSpec((rows, D), lambda i: (i, 0)),
            pl.BlockSpec((rows, D), lambda i: (i, 0)),
        ],
        out_shape=[
            jax.ShapeDtypeStruct((N, HP), jnp.float32),
            jax.ShapeDtypeStruct((N, D), jnp.float32),
            jax.ShapeDtypeStruct((N, D), jnp.float32),
        ],
    )(hpre, st, gamma, beta, wu16, wlin, cp)


def _post2_kern(acc_ref, cnt_ref, bself_ref, b2_ref, out_ref):
    summed = acc_ref[0] + acc_ref[1] + bself_ref[...]
    cnt = cnt_ref[0, :, 0:1] + cnt_ref[1, :, 0:1] + 1.0
    out_ref[...] = summed / jnp.maximum(cnt, 1.0) + b2_ref[...]


def _tc_post2(acc, cnt, bself, b2row):
    nb = 10
    rows = N // nb
    return pl.pallas_call(
        _post2_kern,
        grid=(nb,),
        in_specs=[
            pl.BlockSpec((NC, rows, D), lambda i: (0, i, 0)),
            pl.BlockSpec((NC, rows, HP), lambda i: (0, i, 0)),
            pl.BlockSpec((rows, D), lambda i: (i, 0)),
            pl.BlockSpec((1, D), lambda i: (0, 0)),
        ],
        out_specs=pl.BlockSpec((rows, D), lambda i: (i, 0)),
        out_shape=jax.ShapeDtypeStruct((N, D), jnp.float32),
    )(acc, cnt, bself, b2row)


def kernel(x, edge_index, W1_lin, W1_u, c1, b1, bn_gamma, bn_beta,
           W2_lin, W2_u, c2, b2):
    src = edge_index[0].astype(jnp.int32)
    dst = edge_index[1].astype(jnp.int32)

    w1u16 = jnp.pad(W1_u, ((0, HP - H), (0, 0)))
    w2u16 = jnp.pad(W2_u, ((0, HP - H), (0, 0)))
    c1p = jnp.pad(c1, (0, HP - H), constant_values=NEG).reshape(1, HP)
    c2p = jnp.pad(c2, (0, HP - H), constant_values=NEG).reshape(1, HP)
    w1l_bf = W1_lin.astype(jnp.bfloat16)
    w2l_bf = W2_lin.astype(jnp.bfloat16)
    b1r = b1.reshape(1, D)
    b2r = b2.reshape(1, D)

    a1, bs1 = _tc_prep(x, w1u16, W1_lin, c1p)
    xj1 = _sc_gather_x(x, src)
    l1 = _sc_gather_a(a1, src, dst)
    msg1, wrow = _tc_edge(xj1, l1, w1l_bf, c1p, True)
    acc1, cnt = _sc_scatter(msg1, wrow, dst, True)
    hpre, st = _tc_post1(acc1, cnt, bs1, b1r)
    a2, bs2, h = _tc_bn_prep(hpre, st, bn_gamma.reshape(1, D),
                             bn_beta.reshape(1, D), w2u16, W2_lin, c2p)

    xj2 = _sc_gather_x(h, src)
    l2 = _sc_gather_a(a2, src, dst)
    (msg2,) = _tc_edge(xj2, l2, w2l_bf, c2p, False)
    acc2, _ = _sc_scatter(msg2, wrow, dst, False)
    return _tc_post2(acc2, cnt, bs2, b2r)


# speedup vs baseline: 1.0250x; 1.0237x over previous
import functools

import jax
import jax.numpy as jnp
from jax import lax
from jax.experimental import pallas as pl
from jax.experimental.pallas import tpu as pltpu
from jax.experimental.pallas import tpu_sc as plsc

N = 10000
E = 320000
D = 128
H = 12
HP = 16
NEG = -1e30

NC = 2
NS = 16
NW = NC * NS
E_W = E // NW
KB = 200
NBLK = E_W // KB
KS = 80
NBLKS = E_W // KS
ZR = 40
KE = 1000
GE = E // KE


def _mesh():
    return plsc.VectorSubcoreMesh(core_axis_name="c", subcore_axis_name="s")


def _gather_body(xb_hbm, a_hbm, src_hbm, dst_hbm, xj_hbm, l_hbm,
                 srcv, dstv, xjv, ajv, aiv, lv, sg, so):
    wid = lax.axis_index("s") * NC + lax.axis_index("c")
    base = wid * E_W
    lane = lax.iota(jnp.int32, HP)
    is12 = lane == 12

    def fetch(b, s):
        off = base + b * KB
        pltpu.sync_copy(src_hbm.at[pl.ds(off, KB)], srcv.at[s, pl.ds(0, KB)])
        pltpu.sync_copy(dst_hbm.at[pl.ds(off, KB)], dstv.at[s, pl.ds(0, KB)])
        idx_s = srcv.at[s, pl.ds(0, KB)]
        idx_d = dstv.at[s, pl.ds(0, KB)]
        pltpu.async_copy(xb_hbm.at[idx_s], xjv.at[s], sg.at[s])
        pltpu.async_copy(a_hbm.at[idx_s], ajv.at[s], sg.at[s])
        pltpu.async_copy(a_hbm.at[idx_d], aiv.at[s], sg.at[s])

    def wait_fetch(s):
        idx_s = srcv.at[s, pl.ds(0, KB)]
        idx_d = dstv.at[s, pl.ds(0, KB)]
        pltpu.make_async_copy(xb_hbm.at[idx_s], xjv.at[s], sg.at[s]).wait()
        pltpu.make_async_copy(a_hbm.at[idx_s], ajv.at[s], sg.at[s]).wait()
        pltpu.make_async_copy(a_hbm.at[idx_d], aiv.at[s], sg.at[s]).wait()

    def wait_out(s):
        pltpu.make_async_copy(xjv.at[s], xj_hbm.at[pl.ds(0, KB)], so.at[s]).wait()
        pltpu.make_async_copy(lv.at[s], l_hbm.at[pl.ds(0, KB)], so.at[s]).wait()

    fetch(0, 0)

    def blk(b, _):
        s = lax.rem(b, 2)
        s2 = 1 - s

        @pl.when(b + 1 < NBLK)
        def _():
            @pl.when(b >= 1)
            def _():
                wait_out(s2)
            fetch(b + 1, s2)

        wait_fetch(s)
        srcv_s, dstv_s = srcv.at[s], dstv.at[s]
        aiv_s, ajv_s, lv_s = aiv.at[s], ajv.at[s], lv.at[s]

        def grp16(g, nj):
            sv = srcv_s[pl.ds(g * 16, 16)]
            dv = dstv_s[pl.ds(g * 16, 16)]
            wv = jnp.where(sv != dv, 1.0, 0.0).astype(jnp.float32)
            for j in range(nj):
                e = g * 16 + j
                l = aiv_s[e, :] - ajv_s[e, :]
                lv_s[e, :] = jnp.where(is12, wv[j], l)

        def grp(g, _):
            grp16(g, 16)
            return 0

        lax.fori_loop(0, KB // 16, grp, 0)
        if KB % 16:
            grp16(KB // 16, KB % 16)
        off = base + b * KB
        pltpu.async_copy(xjv.at[s], xj_hbm.at[pl.ds(off, KB)], so.at[s])
        pltpu.async_copy(lv.at[s], l_hbm.at[pl.ds(off, KB)], so.at[s])
        return 0

    lax.fori_loop(0, NBLK, blk, 0)
    wait_out(0)
    wait_out(1)


def _sc_gather(xb, a, src, dst):
    k = pl.kernel(
        _gather_body,
        out_type=(jax.ShapeDtypeStruct((E, D), jnp.float32),
                  jax.ShapeDtypeStruct((E, HP), jnp.float32)),
        mesh=_mesh(),
        compiler_params=pltpu.CompilerParams(use_tc_tiling_on_sc=False),
        scratch_types=[
            pltpu.VMEM((2, KB + 16), jnp.int32),
            pltpu.VMEM((2, KB + 16), jnp.int32),
            pltpu.VMEM((2, KB, D), jnp.float32),
            pltpu.VMEM((2, KB, HP), jnp.float32),
            pltpu.VMEM((2, KB, HP), jnp.float32),
            pltpu.VMEM((2, KB, HP), jnp.float32),
            pltpu.SemaphoreType.DMA((2,)),
            pltpu.SemaphoreType.DMA((2,)),
        ],
    )
    return k(xb, a, src, dst)


def _scatter_body(with_cnt, msg_hbm, wrow_hbm, dst_hbm, acc_out, cnt_out,
                  msgv, dstv, wrowv, zbuf, zbufc, acc_sh, cnt_sh, sl):
    cid = lax.axis_index("c")
    sid = lax.axis_index("s")
    wid = sid * NC + cid
    base = wid * E_W

    def zrow(i, _):
        r = i // (D // HP)
        c = i % (D // HP)
        zbuf[r, pl.ds(c * HP, HP)] = jnp.zeros((HP,), jnp.float32)
        return 0

    lax.fori_loop(0, ZR * (D // HP), zrow, 0)

    if with_cnt:
        def zrowc(i, _):
            zbufc[i, :] = jnp.zeros((HP,), jnp.float32)
            return 0
        lax.fori_loop(0, ZR, zrowc, 0)

    nzb = N // ZR

    def zcp(k, _):
        b = sid + k * NS

        @pl.when(b < nzb)
        def _():
            pltpu.sync_copy(zbuf, acc_sh.at[pl.ds(b * ZR, ZR)])
            if with_cnt:
                pltpu.sync_copy(zbufc, cnt_sh.at[pl.ds(b * ZR, ZR)])
        return 0

    lax.fori_loop(0, (nzb + NS - 1) // NS, zcp, 0)
    plsc.subcore_barrier()

    def fetch(b, s):
        off = base + b * KS
        pltpu.async_copy(msg_hbm.at[pl.ds(off, KS)], msgv.at[s], sl.at[s])
        pltpu.async_copy(dst_hbm.at[pl.ds(off, KS)], dstv.at[s], sl.at[s])
        if with_cnt:
            pltpu.async_copy(wrow_hbm.at[pl.ds(off, KS)], wrowv.at[s], sl.at[s])

    def wait_fetch(s):
        pltpu.make_async_copy(msg_hbm.at[pl.ds(0, KS)], msgv.at[s], sl.at[s]).wait()
        pltpu.make_async_copy(dst_hbm.at[pl.ds(0, KS)], dstv.at[s], sl.at[s]).wait()
        if with_cnt:
            pltpu.make_async_copy(wrow_hbm.at[pl.ds(0, KS)], wrowv.at[s], sl.at[s]).wait()

    fetch(0, 0)

    def blk(b, _):
        s = lax.rem(b, 2)

        @pl.when(b + 1 < NBLKS)
        def _():
            fetch(b + 1, 1 - s)

        wait_fetch(s)
        pltpu.sync_copy(msgv.at[s], acc_sh.at[dstv.at[s]], add=True)
        if with_cnt:
            pltpu.sync_copy(wrowv.at[s], cnt_sh.at[dstv.at[s]], add=True)
        return 0

    lax.fori_loop(0, NBLKS, blk, 0)
    plsc.subcore_barrier()

    @pl.when(sid == 0)
    def _():
        pltpu.sync_copy(acc_sh, acc_out.at[cid])
        if with_cnt:
            pltpu.sync_copy(cnt_sh, cnt_out.at[cid])


def _sc_scatter(msg, wrow, dst, with_cnt):
    k = pl.kernel(
        functools.partial(_scatter_body, with_cnt),
        out_type=(jax.ShapeDtypeStruct((NC, N, D), jnp.float32),
                  jax.ShapeDtypeStruct((NC, N, HP), jnp.float32)),
        mesh=_mesh(),
        compiler_params=pltpu.CompilerParams(use_tc_tiling_on_sc=False),
        scratch_types=[
            pltpu.VMEM((2, KS, D), jnp.float32),
            pltpu.VMEM((2, KS), jnp.int32),
            pltpu.VMEM((2, KS, HP), jnp.float32),
            pltpu.VMEM((ZR, D), jnp.float32),
            pltpu.VMEM((ZR, HP), jnp.float32),
            pltpu.VMEM_SHARED((N, D), jnp.float32),
            pltpu.VMEM_SHARED((N, HP), jnp.float32),
            pltpu.SemaphoreType.DMA((2,)),
        ],
    )
    return k(msg, wrow, dst)


def _wc_from(wl, cp):
    m = jnp.max(cp, axis=1, keepdims=True)
    p = jnp.exp(cp - m)
    sc = p / jnp.sum(p, axis=1, keepdims=True)
    wc = jnp.zeros((D, D), jnp.float32)
    for h in range(H):
        wc = wc + sc[0, h] * wl[h * D:(h + 1) * D, :]
    return wc


def _prep_kern(x_ref, wu_ref, wl_ref, cp_ref, a_ref, b_ref):
    x = x_ref[...]
    a_ref[...] = lax.dot_general(x, wu_ref[...], (((1,), (1,)), ((), ())),
                                 preferred_element_type=jnp.float32)
    wc = _wc_from(wl_ref[...], cp_ref[...])
    b_ref[...] = lax.dot_general(x, wc, (((1,), (1,)), ((), ())),
                                 preferred_element_type=jnp.float32)


def _tc_prep(x, wu16, wlin, cp):
    nb = 10
    rows = N // nb
    return pl.pallas_call(
        _prep_kern,
        grid=(nb,),
        in_specs=[
            pl.BlockSpec((rows, D), lambda i: (i, 0)),
            pl.BlockSpec((HP, D), lambda i: (0, 0)),
            pl.BlockSpec((H * D, D), lambda i: (0, 0)),
            pl.BlockSpec((1, HP), lambda i: (0, 0)),
        ],
        out_specs=[
            pl.BlockSpec((rows, HP), lambda i: (i, 0)),
            pl.BlockSpec((rows, D), lambda i: (i, 0)),
        ],
        out_shape=[
            jax.ShapeDtypeStruct((N, HP), jnp.float32),
            jax.ShapeDtypeStruct((N, D), jnp.float32),
        ],
    )(x, wu16, wlin, cp)


def _edge_kern(with_w, xj_ref, l_ref, wl_ref, cp_ref, msg_ref, *rest):
    l = l_ref[...]
    logits = l + cp_ref[...]
    m = jnp.max(logits, axis=1, keepdims=True)
    p = jnp.exp(logits - m)
    att = p / jnp.sum(p, axis=1, keepdims=True)
    w = l[:, 12:13]
    attw = att * w
    xj = xj_ref[...].astype(jnp.bfloat16)
    acc = jnp.zeros((KE, D), jnp.float32)
    for h in range(H):
        xt = lax.dot_general(xj, wl_ref[pl.ds(h * D, D), :],
                             (((1,), (1,)), ((), ())),
                             preferred_element_type=jnp.float32)
        acc = acc + attw[:, h:h + 1] * xt
    msg_ref[...] = acc
    if with_w:
        rest[0][...] = jnp.broadcast_to(w, (KE, HP))


def _tc_edge(xj, l, wlin_bf, cp, with_w):
    out_specs = [pl.BlockSpec((KE, D), lambda i: (i, 0))]
    out_shape = [jax.ShapeDtypeStruct((E, D), jnp.float32)]
    if with_w:
        out_specs.append(pl.BlockSpec((KE, HP), lambda i: (i, 0)))
        out_shape.append(jax.ShapeDtypeStruct((E, HP), jnp.float32))
    return pl.pallas_call(
        functools.partial(_edge_kern, with_w),
        grid=(GE,),
        in_specs=[
            pl.BlockSpec((KE, D), lambda i: (i, 0)),
            pl.BlockSpec((KE, HP), lambda i: (i, 0)),
            pl.BlockSpec((H * D, D), lambda i: (0, 0)),
            pl.BlockSpec((1, HP), lambda i: (0, 0)),
        ],
        out_specs=out_specs,
        out_shape=out_shape,
    )(xj, l, wlin_bf, cp)


def _post1_kern(acc_ref, cnt_ref, bself_ref, b1_ref, hpre_ref, st_ref):
    i = pl.program_id(0)
    summed = acc_ref[0] + acc_ref[1] + bself_ref[...]
    cnt = cnt_ref[0, :, 0:1] + cnt_ref[1, :, 0:1] + 1.0
    hpre = summed / jnp.maximum(cnt, 1.0) + b1_ref[...]
    hpre = jnp.maximum(hpre, 0.0)
    hpre_ref[...] = hpre

    @pl.when(i == 0)
    def _():
        st_ref[...] = jnp.zeros_like(st_ref)

    s1 = jnp.sum(hpre, axis=0, keepdims=True)
    s2 = jnp.sum(hpre * hpre, axis=0, keepdims=True)
    st_ref[0:1, :] += s1
    st_ref[1:2, :] += s2


def _tc_post1(acc, cnt, bself, b1row):
    nb = 10
    rows = N // nb
    return pl.pallas_call(
        _post1_kern,
        grid=(nb,),
        in_specs=[
            pl.BlockSpec((NC, rows, D), lambda i: (0, i, 0)),
            pl.BlockSpec((NC, rows, HP), lambda i: (0, i, 0)),
            pl.BlockSpec((rows, D), lambda i: (i, 0)),
            pl.BlockSpec((1, D), lambda i: (0, 0)),
        ],
        out_specs=[
            pl.BlockSpec((rows, D), lambda i: (i, 0)),
            pl.BlockSpec((8, D), lambda i: (0, 0)),
        ],
        out_shape=[
            jax.ShapeDtypeStruct((N, D), jnp.float32),
            jax.ShapeDtypeStruct((8, D), jnp.float32),
        ],
    )(acc, cnt, bself, b1row)


def _bn_kern(hpre_ref, st_ref, g_ref, bt_ref, wu_ref, wl_ref, cp_ref,
             a_ref, b_ref, hb_ref):
    mean = st_ref[0:1, :] / float(N)
    var = st_ref[1:2, :] / float(N) - mean * mean
    h = (hpre_ref[...] - mean) * lax.rsqrt(var + 1e-5) * g_ref[...] + bt_ref[...]
    a_ref[...] = lax.dot_general(h, wu_ref[...], (((1,), (1,)), ((), ())),
                                 preferred_element_type=jnp.float32)
    wc = _wc_from(wl_ref[...], cp_ref[...])
    b_ref[...] = lax.dot_general(h, wc, (((1,), (1,)), ((), ())),
                                 preferred_element_type=jnp.float32)
    hb_ref[...] = h


def _tc_bn_prep(hpre, st, gamma, beta, wu16, wlin, cp):
    nb = 10
    rows = N // nb
    return pl.pallas_call(
        _bn_kern,
        grid=(nb,),
        in_specs=[
            pl.BlockSpec((rows, D), lambda i: (i, 0)),
            pl.BlockSpec((8, D), lambda i: (0, 0)),
            pl.BlockSpec((1, D), lambda i: (0, 0)),
            pl.BlockSpec((1, D), lambda i: (0, 0)),
            pl.BlockSpec((HP, D), lambda i: (0, 0)),
            pl.BlockSpec((H * D, D), lambda i: (0, 0)),
            pl.BlockSpec((1, HP), lambda i: (0, 0)),
        ],
        out_specs=[
            pl.BlockSpec((rows, HP), lambda i: (i, 0)),
            pl.BlockSpec((rows, D), lambda i: (i, 0)),
            pl.BlockSpec((rows, D), lambda i: (i, 0)),
        ],
        out_shape=[
            jax.ShapeDtypeStruct((N, HP), jnp.float32),
            jax.ShapeDtypeStruct((N, D), jnp.float32),
            jax.ShapeDtypeStruct((N, D), jnp.float32),
        ],
    )(hpre, st, gamma, beta, wu16, wlin, cp)


def _post2_kern(acc_ref, cnt_ref, bself_ref, b2_ref, out_ref):
    summed = acc_ref[0] + acc_ref[1] + bself_ref[...]
    cnt = cnt_ref[0, :, 0:1] + cnt_ref[1, :, 0:1] + 1.0
    out_ref[...] = summed / jnp.maximum(cnt, 1.0) + b2_ref[...]


def _tc_post2(acc, cnt, bself, b2row):
    nb = 10
    rows = N // nb
    return pl.pallas_call(
        _post2_kern,
        grid=(nb,),
        in_specs=[
            pl.BlockSpec((NC, rows, D), lambda i: (0, i, 0)),
            pl.BlockSpec((NC, rows, HP), lambda i: (0, i, 0)),
            pl.BlockSpec((rows, D), lambda i: (i, 0)),
            pl.BlockSpec((1, D), lambda i: (0, 0)),
        ],
        out_specs=pl.BlockSpec((rows, D), lambda i: (i, 0)),
        out_shape=jax.ShapeDtypeStruct((N, D), jnp.float32),
    )(acc, cnt, bself, b2row)


def kernel(x, edge_index, W1_lin, W1_u, c1, b1, bn_gamma, bn_beta,
           W2_lin, W2_u, c2, b2):
    src = edge_index[0].astype(jnp.int32)
    dst = edge_index[1].astype(jnp.int32)

    w1u16 = jnp.pad(W1_u, ((0, HP - H), (0, 0)))
    w2u16 = jnp.pad(W2_u, ((0, HP - H), (0, 0)))
    c1p = jnp.pad(c1, (0, HP - H), constant_values=NEG).reshape(1, HP)
    c2p = jnp.pad(c2, (0, HP - H), constant_values=NEG).reshape(1, HP)
    w1l_bf = W1_lin.astype(jnp.bfloat16)
    w2l_bf = W2_lin.astype(jnp.bfloat16)
    b1r = b1.reshape(1, D)
    b2r = b2.reshape(1, D)

    a1, bs1 = _tc_prep(x, w1u16, W1_lin, c1p)
    xj1, l1 = _sc_gather(x, a1, src, dst)
    msg1, wrow = _tc_edge(xj1, l1, w1l_bf, c1p, True)
    acc1, cnt = _sc_scatter(msg1, wrow, dst, True)
    hpre, st = _tc_post1(acc1, cnt, bs1, b1r)
    a2, bs2, h = _tc_bn_prep(hpre, st, bn_gamma.reshape(1, D),
                             bn_beta.reshape(1, D), w2u16, W2_lin, c2p)

    xj2, l2 = _sc_gather(h, a2, src, dst)
    (msg2,) = _tc_edge(xj2, l2, w2l_bf, c2p, False)
    acc2, _ = _sc_scatter(msg2, wrow, dst, False)
    return _tc_post2(acc2, cnt, bs2, b2r)


# speedup vs baseline: 1.0524x; 1.0267x over previous
import functools

import jax
import jax.numpy as jnp
from jax import lax
from jax.experimental import pallas as pl
from jax.experimental.pallas import tpu as pltpu
from jax.experimental.pallas import tpu_sc as plsc

N = 10000
E = 320000
D = 128
H = 12
HP = 16
NEG = -1e30

NC = 2
NS = 16
NW = NC * NS
E_W = E // NW
KB = 200
NBLK = E_W // KB
KS = 80
NBLKS = E_W // KS
ZR = 40
KE = 2000
GE = E // KE


def _mesh():
    return plsc.VectorSubcoreMesh(core_axis_name="c", subcore_axis_name="s")


def _gather_body(xb_hbm, a_hbm, src_hbm, dst_hbm, xj_hbm, l_hbm,
                 srcv, dstv, xjv, ajv, aiv, lv, sg, so):
    wid = lax.axis_index("s") * NC + lax.axis_index("c")
    base = wid * E_W
    lane = lax.iota(jnp.int32, HP)
    is12 = lane == 12

    def fetch(b, s):
        off = base + b * KB
        pltpu.sync_copy(src_hbm.at[pl.ds(off, KB)], srcv.at[s, pl.ds(0, KB)])
        pltpu.sync_copy(dst_hbm.at[pl.ds(off, KB)], dstv.at[s, pl.ds(0, KB)])
        idx_s = srcv.at[s, pl.ds(0, KB)]
        idx_d = dstv.at[s, pl.ds(0, KB)]
        pltpu.async_copy(xb_hbm.at[idx_s], xjv.at[s], sg.at[s])
        pltpu.async_copy(a_hbm.at[idx_s], ajv.at[s], sg.at[s])
        pltpu.async_copy(a_hbm.at[idx_d], aiv.at[s], sg.at[s])

    def wait_fetch(s):
        idx_s = srcv.at[s, pl.ds(0, KB)]
        idx_d = dstv.at[s, pl.ds(0, KB)]
        pltpu.make_async_copy(xb_hbm.at[idx_s], xjv.at[s], sg.at[s]).wait()
        pltpu.make_async_copy(a_hbm.at[idx_s], ajv.at[s], sg.at[s]).wait()
        pltpu.make_async_copy(a_hbm.at[idx_d], aiv.at[s], sg.at[s]).wait()

    def wait_out(s):
        pltpu.make_async_copy(xjv.at[s], xj_hbm.at[pl.ds(0, KB)], so.at[s]).wait()
        pltpu.make_async_copy(lv.at[s], l_hbm.at[pl.ds(0, KB)], so.at[s]).wait()

    fetch(0, 0)

    def blk(b, _):
        s = lax.rem(b, 2)
        s2 = 1 - s

        @pl.when(b + 1 < NBLK)
        def _():
            @pl.when(b >= 1)
            def _():
                wait_out(s2)
            fetch(b + 1, s2)

        wait_fetch(s)
        srcv_s, dstv_s = srcv.at[s], dstv.at[s]
        aiv_s, ajv_s, lv_s = aiv.at[s], ajv.at[s], lv.at[s]

        def grp16(g, nj):
            sv = srcv_s[pl.ds(g * 16, 16)]
            dv = dstv_s[pl.ds(g * 16, 16)]
            wv = jnp.where(sv != dv, 1.0, 0.0).astype(jnp.float32)
            for j in range(nj):
                e = g * 16 + j
                l = aiv_s[e, :] - ajv_s[e, :]
                lv_s[e, :] = jnp.where(is12, wv[j], l)

        def grp(g, _):
            grp16(g, 16)
            return 0

        lax.fori_loop(0, KB // 16, grp, 0)
        if KB % 16:
            grp16(KB // 16, KB % 16)
        off = base + b * KB
        pltpu.async_copy(xjv.at[s], xj_hbm.at[pl.ds(off, KB)], so.at[s])
        pltpu.async_copy(lv.at[s], l_hbm.at[pl.ds(off, KB)], so.at[s])
        return 0

    lax.fori_loop(0, NBLK, blk, 0)
    wait_out(0)
    wait_out(1)


def _sc_gather(xb, a, src, dst):
    k = pl.kernel(
        _gather_body,
        out_type=(jax.ShapeDtypeStruct((E, D), jnp.float32),
                  jax.ShapeDtypeStruct((E, HP), jnp.float32)),
        mesh=_mesh(),
        compiler_params=pltpu.CompilerParams(use_tc_tiling_on_sc=False),
        scratch_types=[
            pltpu.VMEM((2, KB + 16), jnp.int32),
            pltpu.VMEM((2, KB + 16), jnp.int32),
            pltpu.VMEM((2, KB, D), jnp.float32),
            pltpu.VMEM((2, KB, HP), jnp.float32),
            pltpu.VMEM((2, KB, HP), jnp.float32),
            pltpu.VMEM((2, KB, HP), jnp.float32),
            pltpu.SemaphoreType.DMA((2,)),
            pltpu.SemaphoreType.DMA((2,)),
        ],
    )
    return k(xb, a, src, dst)


def _scatter_body(with_cnt, msg_hbm, wrow_hbm, dst_hbm, acc_out, cnt_out,
                  msgv, dstv, wrowv, zbuf, zbufc, acc_sh, cnt_sh, sl):
    cid = lax.axis_index("c")
    sid = lax.axis_index("s")
    wid = sid * NC + cid
    base = wid * E_W

    def zrow(i, _):
        r = i // (D // HP)
        c = i % (D // HP)
        zbuf[r, pl.ds(c * HP, HP)] = jnp.zeros((HP,), jnp.float32)
        return 0

    lax.fori_loop(0, ZR * (D // HP), zrow, 0)

    if with_cnt:
        def zrowc(i, _):
            zbufc[i, :] = jnp.zeros((HP,), jnp.float32)
            return 0
        lax.fori_loop(0, ZR, zrowc, 0)

    nzb = N // ZR

    def zcp(k, _):
        b = sid + k * NS

        @pl.when(b < nzb)
        def _():
            pltpu.sync_copy(zbuf, acc_sh.at[pl.ds(b * ZR, ZR)])
            if with_cnt:
                pltpu.sync_copy(zbufc, cnt_sh.at[pl.ds(b * ZR, ZR)])
        return 0

    lax.fori_loop(0, (nzb + NS - 1) // NS, zcp, 0)
    plsc.subcore_barrier()

    def fetch(b, s):
        off = base + b * KS
        pltpu.async_copy(msg_hbm.at[pl.ds(off, KS)], msgv.at[s], sl.at[s])
        pltpu.async_copy(dst_hbm.at[pl.ds(off, KS)], dstv.at[s], sl.at[s])
        if with_cnt:
            pltpu.async_copy(wrow_hbm.at[pl.ds(off, KS)], wrowv.at[s], sl.at[s])

    def wait_fetch(s):
        pltpu.make_async_copy(msg_hbm.at[pl.ds(0, KS)], msgv.at[s], sl.at[s]).wait()
        pltpu.make_async_copy(dst_hbm.at[pl.ds(0, KS)], dstv.at[s], sl.at[s]).wait()
        if with_cnt:
            pltpu.make_async_copy(wrow_hbm.at[pl.ds(0, KS)], wrowv.at[s], sl.at[s]).wait()

    fetch(0, 0)

    def blk(b, _):
        s = lax.rem(b, 2)

        @pl.when(b + 1 < NBLKS)
        def _():
            fetch(b + 1, 1 - s)

        wait_fetch(s)
        pltpu.sync_copy(msgv.at[s], acc_sh.at[dstv.at[s]], add=True)
        if with_cnt:
            pltpu.sync_copy(wrowv.at[s], cnt_sh.at[dstv.at[s]], add=True)
        return 0

    lax.fori_loop(0, NBLKS, blk, 0)
    plsc.subcore_barrier()

    @pl.when(sid == 0)
    def _():
        pltpu.sync_copy(acc_sh, acc_out.at[cid])
        if with_cnt:
            pltpu.sync_copy(cnt_sh, cnt_out.at[cid])


def _sc_scatter(msg, wrow, dst, with_cnt):
    k = pl.kernel(
        functools.partial(_scatter_body, with_cnt),
        out_type=(jax.ShapeDtypeStruct((NC, N, D), jnp.float32),
                  jax.ShapeDtypeStruct((NC, N, HP), jnp.float32)),
        mesh=_mesh(),
        compiler_params=pltpu.CompilerParams(use_tc_tiling_on_sc=False),
        scratch_types=[
            pltpu.VMEM((2, KS, D), jnp.float32),
            pltpu.VMEM((2, KS), jnp.int32),
            pltpu.VMEM((2, KS, HP), jnp.float32),
            pltpu.VMEM((ZR, D), jnp.float32),
            pltpu.VMEM((ZR, HP), jnp.float32),
            pltpu.VMEM_SHARED((N, D), jnp.float32),
            pltpu.VMEM_SHARED((N, HP), jnp.float32),
            pltpu.SemaphoreType.DMA((2,)),
        ],
    )
    return k(msg, wrow, dst)


def _wc_from(wl, cp):
    m = jnp.max(cp, axis=1, keepdims=True)
    p = jnp.exp(cp - m)
    sc = p / jnp.sum(p, axis=1, keepdims=True)
    wc = jnp.zeros((D, D), jnp.float32)
    for h in range(H):
        wc = wc + sc[0, h] * wl[h * D:(h + 1) * D, :]
    return wc


def _prep_kern(x_ref, wu_ref, wl_ref, cp_ref, a_ref, b_ref):
    x = x_ref[...]
    a_ref[...] = lax.dot_general(x, wu_ref[...], (((1,), (1,)), ((), ())),
                                 preferred_element_type=jnp.float32)
    wc = _wc_from(wl_ref[...], cp_ref[...])
    b_ref[...] = lax.dot_general(x, wc, (((1,), (1,)), ((), ())),
                                 preferred_element_type=jnp.float32)


def _tc_prep(x, wu16, wlin, cp):
    nb = 10
    rows = N // nb
    return pl.pallas_call(
        _prep_kern,
        grid=(nb,),
        in_specs=[
            pl.BlockSpec((rows, D), lambda i: (i, 0)),
            pl.BlockSpec((HP, D), lambda i: (0, 0)),
            pl.BlockSpec((H * D, D), lambda i: (0, 0)),
            pl.BlockSpec((1, HP), lambda i: (0, 0)),
        ],
        out_specs=[
            pl.BlockSpec((rows, HP), lambda i: (i, 0)),
            pl.BlockSpec((rows, D), lambda i: (i, 0)),
        ],
        out_shape=[
            jax.ShapeDtypeStruct((N, HP), jnp.float32),
            jax.ShapeDtypeStruct((N, D), jnp.float32),
        ],
    )(x, wu16, wlin, cp)


def _edge_kern(with_w, xj_ref, l_ref, wl_ref, cp_ref, msg_ref, *rest):
    l = l_ref[...]
    logits = l + cp_ref[...]
    m = jnp.max(logits, axis=1, keepdims=True)
    p = jnp.exp(logits - m)
    att = p / jnp.sum(p, axis=1, keepdims=True)
    w = l[:, 12:13]
    attw = att * w
    xj = xj_ref[...].astype(jnp.bfloat16)
    acc = jnp.zeros((KE, D), jnp.float32)
    for h in range(H):
        xt = lax.dot_general(xj, wl_ref[pl.ds(h * D, D), :],
                             (((1,), (1,)), ((), ())),
                             preferred_element_type=jnp.float32)
        acc = acc + attw[:, h:h + 1] * xt
    msg_ref[...] = acc
    if with_w:
        rest[0][...] = jnp.broadcast_to(w, (KE, HP))


def _tc_edge(xj, l, wlin_bf, cp, with_w):
    out_specs = [pl.BlockSpec((KE, D), lambda i: (i, 0))]
    out_shape = [jax.ShapeDtypeStruct((E, D), jnp.float32)]
    if with_w:
        out_specs.append(pl.BlockSpec((KE, HP), lambda i: (i, 0)))
        out_shape.append(jax.ShapeDtypeStruct((E, HP), jnp.float32))
    return pl.pallas_call(
        functools.partial(_edge_kern, with_w),
        grid=(GE,),
        in_specs=[
            pl.BlockSpec((KE, D), lambda i: (i, 0)),
            pl.BlockSpec((KE, HP), lambda i: (i, 0)),
            pl.BlockSpec((H * D, D), lambda i: (0, 0)),
            pl.BlockSpec((1, HP), lambda i: (0, 0)),
        ],
        out_specs=out_specs,
        out_shape=out_shape,
    )(xj, l, wlin_bf, cp)


def _post1_kern(acc_ref, cnt_ref, bself_ref, b1_ref, hpre_ref, st_ref):
    i = pl.program_id(0)
    summed = acc_ref[0] + acc_ref[1] + bself_ref[...]
    cnt = cnt_ref[0, :, 0:1] + cnt_ref[1, :, 0:1] + 1.0
    hpre = summed / jnp.maximum(cnt, 1.0) + b1_ref[...]
    hpre = jnp.maximum(hpre, 0.0)
    hpre_ref[...] = hpre

    @pl.when(i == 0)
    def _():
        st_ref[...] = jnp.zeros_like(st_ref)

    s1 = jnp.sum(hpre, axis=0, keepdims=True)
    s2 = jnp.sum(hpre * hpre, axis=0, keepdims=True)
    st_ref[0:1, :] += s1
    st_ref[1:2, :] += s2


def _tc_post1(acc, cnt, bself, b1row):
    nb = 10
    rows = N // nb
    return pl.pallas_call(
        _post1_kern,
        grid=(nb,),
        in_specs=[
            pl.BlockSpec((NC, rows, D), lambda i: (0, i, 0)),
            pl.BlockSpec((NC, rows, HP), lambda i: (0, i, 0)),
            pl.BlockSpec((rows, D), lambda i: (i, 0)),
            pl.BlockSpec((1, D), lambda i: (0, 0)),
        ],
        out_specs=[
            pl.BlockSpec((rows, D), lambda i: (i, 0)),
            pl.BlockSpec((8, D), lambda i: (0, 0)),
        ],
        out_shape=[
            jax.ShapeDtypeStruct((N, D), jnp.float32),
            jax.ShapeDtypeStruct((8, D), jnp.float32),
        ],
    )(acc, cnt, bself, b1row)


def _bn_kern(hpre_ref, st_ref, g_ref, bt_ref, wu_ref, wl_ref, cp_ref,
             a_ref, b_ref, hb_ref):
    mean = st_ref[0:1, :] / float(N)
    var = st_ref[1:2, :] / float(N) - mean * mean
    h = (hpre_ref[...] - mean) * lax.rsqrt(var + 1e-5) * g_ref[...] + bt_ref[...]
    a_ref[...] = lax.dot_general(h, wu_ref[...], (((1,), (1,)), ((), ())),
                                 preferred_element_type=jnp.float32)
    wc = _wc_from(wl_ref[...], cp_ref[...])
    b_ref[...] = lax.dot_general(h, wc, (((1,), (1,)), ((), ())),
                                 preferred_element_type=jnp.float32)
    hb_ref[...] = h


def _tc_bn_prep(hpre, st, gamma, beta, wu16, wlin, cp):
    nb = 10
    rows = N // nb
    return pl.pallas_call(
        _bn_kern,
        grid=(nb,),
        in_specs=[
            pl.BlockSpec((rows, D), lambda i: (i, 0)),
            pl.BlockSpec((8, D), lambda i: (0, 0)),
            pl.BlockSpec((1, D), lambda i: (0, 0)),
            pl.BlockSpec((1, D), lambda i: (0, 0)),
            pl.BlockSpec((HP, D), lambda i: (0, 0)),
            pl.BlockSpec((H * D, D), lambda i: (0, 0)),
            pl.BlockSpec((1, HP), lambda i: (0, 0)),
        ],
        out_specs=[
            pl.BlockSpec((rows, HP), lambda i: (i, 0)),
            pl.BlockSpec((rows, D), lambda i: (i, 0)),
            pl.BlockSpec((rows, D), lambda i: (i, 0)),
        ],
        out_shape=[
            jax.ShapeDtypeStruct((N, HP), jnp.float32),
            jax.ShapeDtypeStruct((N, D), jnp.float32),
            jax.ShapeDtypeStruct((N, D), jnp.float32),
        ],
    )(hpre, st, gamma, beta, wu16, wlin, cp)


def _post2_kern(acc_ref, cnt_ref, bself_ref, b2_ref, out_ref):
    summed = acc_ref[0] + acc_ref[1] + bself_ref[...]
    cnt = cnt_ref[0, :, 0:1] + cnt_ref[1, :, 0:1] + 1.0
    out_ref[...] = summed / jnp.maximum(cnt, 1.0) + b2_ref[...]


def _tc_post2(acc, cnt, bself, b2row):
    nb = 10
    rows = N // nb
    return pl.pallas_call(
        _post2_kern,
        grid=(nb,),
        in_specs=[
            pl.BlockSpec((NC, rows, D), lambda i: (0, i, 0)),
            pl.BlockSpec((NC, rows, HP), lambda i: (0, i, 0)),
            pl.BlockSpec((rows, D), lambda i: (i, 0)),
            pl.BlockSpec((1, D), lambda i: (0, 0)),
        ],
        out_specs=pl.BlockSpec((rows, D), lambda i: (i, 0)),
        out_shape=jax.ShapeDtypeStruct((N, D), jnp.float32),
    )(acc, cnt, bself, b2row)


def kernel(x, edge_index, W1_lin, W1_u, c1, b1, bn_gamma, bn_beta,
           W2_lin, W2_u, c2, b2):
    src = edge_index[0].astype(jnp.int32)
    dst = edge_index[1].astype(jnp.int32)

    w1u16 = jnp.pad(W1_u, ((0, HP - H), (0, 0)))
    w2u16 = jnp.pad(W2_u, ((0, HP - H), (0, 0)))
    c1p = jnp.pad(c1, (0, HP - H), constant_values=NEG).reshape(1, HP)
    c2p = jnp.pad(c2, (0, HP - H), constant_values=NEG).reshape(1, HP)
    w1l_bf = W1_lin.astype(jnp.bfloat16)
    w2l_bf = W2_lin.astype(jnp.bfloat16)
    b1r = b1.reshape(1, D)
    b2r = b2.reshape(1, D)

    a1, bs1 = _tc_prep(x, w1u16, W1_lin, c1p)
    xj1, l1 = _sc_gather(x, a1, src, dst)
    msg1, wrow = _tc_edge(xj1, l1, w1l_bf, c1p, True)
    acc1, cnt = _sc_scatter(msg1, wrow, dst, True)
    hpre, st = _tc_post1(acc1, cnt, bs1, b1r)
    a2, bs2, h = _tc_bn_prep(hpre, st, bn_gamma.reshape(1, D),
                             bn_beta.reshape(1, D), w2u16, W2_lin, c2p)

    xj2, l2 = _sc_gather(h, a2, src, dst)
    (msg2,) = _tc_edge(xj2, l2, w2l_bf, c2p, False)
    acc2, _ = _sc_scatter(msg2, wrow, dst, False)
    return _tc_post2(acc2, cnt, bs2, b2r)
